# Initial kernel scaffold; baseline (speedup 1.0000x reference)
import jax
import jax.numpy as jnp
from jax.experimental import pallas as pl


def kernel(h, edge_index, e, A_W, A_b, B_W, B_b, gamma, beta):
    raise NotImplementedError("write your pallas kernel here")


# speedup vs baseline: 3.1468x; 3.1468x over previous
import functools

import jax
import jax.numpy as jnp
from jax import lax
from jax.experimental import pallas as pl
from jax.experimental.pallas import tpu as pltpu
from jax.experimental.pallas import tpu_sc as plsc

N = 10000
E = 320000
D = 128

NC = 2
NS = 16
NW = NC * NS

CHUNK = 128
EPW = 10240
PAD_E = EPW * NW
ACC_ROWS = 10240
ROWS_PER_TILE = ACC_ROWS // NS
CP = 80

MM_BLK = 1000


def _mm_body(h_ref, aw_ref, ab_ref, bw_ref, bb_ref, ah_ref, bh_ref):
    hb = h_ref[...]
    ah_ref[...] = jnp.dot(hb, aw_ref[...],
                          preferred_element_type=jnp.float32) + ab_ref[...]
    bh_ref[...] = jnp.dot(hb, bw_ref[...],
                          preferred_element_type=jnp.float32) + bb_ref[...]


def _matmuls(h, A_W, A_b2, B_W, B_b2):
    grid = (N // MM_BLK,)
    full = pl.BlockSpec((D, D), lambda i: (0, 0))
    vec = pl.BlockSpec((1, D), lambda i: (0, 0))
    blk = pl.BlockSpec((MM_BLK, D), lambda i: (i, 0))
    return pl.pallas_call(
        _mm_body,
        grid=grid,
        in_specs=[blk, full, vec, full, vec],
        out_specs=[blk, blk],
        out_shape=[jax.ShapeDtypeStruct((N, D), jnp.float32)] * 2,
    )(h, A_W, A_b2, B_W, B_b2)


def _sc_body(bh, srcp, dstp, out, sidx, didx, rows, cbuf, acc):
    c = lax.axis_index("c")
    s = lax.axis_index("s")
    wid = c * NS + s

    def _zb(i, _):
        cbuf[i // 8, pl.ds((i % 8) * 16, 16)] = jnp.zeros((16,), jnp.float32)
        return _
    lax.fori_loop(0, CP * 8, _zb, None)

    def _z(j, _):
        pltpu.sync_copy(cbuf, acc.at[pl.ds(s * ROWS_PER_TILE + j * CP, CP)])
        return _
    lax.fori_loop(0, ROWS_PER_TILE // CP, _z, None)

    plsc.subcore_barrier()

    base = wid * EPW

    def _edge(k, _):
        off = base + k * CHUNK
        pltpu.sync_copy(srcp.at[pl.ds(off, CHUNK)], sidx)
        pltpu.sync_copy(dstp.at[pl.ds(off, CHUNK)], didx)
        pltpu.sync_copy(bh.at[sidx], rows)
        pltpu.sync_copy(rows, acc.at[didx], add=True)
        return _
    lax.fori_loop(0, EPW // CHUNK, _edge, None)

    plsc.subcore_barrier()

    def _cp(j, _):
        r0 = s * ROWS_PER_TILE + j * CP

        @pl.when(r0 < N)
        def _():
            pltpu.sync_copy(acc.at[pl.ds(r0, CP)], cbuf)
            pltpu.sync_copy(cbuf, out.at[c, pl.ds(r0, CP)])
        return _
    lax.fori_loop(0, ROWS_PER_TILE // CP, _cp, None)


def _sc_aggregate(bh, src_p, dst_p):
    mesh = plsc.VectorSubcoreMesh(core_axis_name="c", subcore_axis_name="s")
    fn = pl.kernel(
        _sc_body,
        out_type=jax.ShapeDtypeStruct((NC, N, D), jnp.float32),
        mesh=mesh,
        scratch_types=[
            pltpu.VMEM((CHUNK,), jnp.int32),
            pltpu.VMEM((CHUNK,), jnp.int32),
            pltpu.VMEM((CHUNK, D), jnp.float32),
            pltpu.VMEM((CP, D), jnp.float32),
            pltpu.VMEM_SHARED((ACC_ROWS, D), jnp.float32),
        ],
    )
    return fn(bh, src_p, dst_p)


def _stats_body(ah_ref, p_ref, pre_ref, st_ref):
    i = pl.program_id(0)
    x = ah_ref[...] + p_ref[0] + p_ref[1]
    pre_ref[...] = x
    st = jnp.concatenate([jnp.sum(x, axis=0, keepdims=True),
                          jnp.sum(x * x, axis=0, keepdims=True)], axis=0)

    @pl.when(i == 0)
    def _():
        st_ref[...] = jnp.zeros_like(st_ref)

    st_ref[...] += st


def _combine_stats(ah, parts):
    grid = (N // MM_BLK,)
    blk = pl.BlockSpec((MM_BLK, D), lambda i: (i, 0))
    pblk = pl.BlockSpec((NC, MM_BLK, D), lambda i: (0, i, 0))
    sblk = pl.BlockSpec((2, D), lambda i: (0, 0))
    return pl.pallas_call(
        _stats_body,
        grid=grid,
        in_specs=[blk, pblk],
        out_specs=[blk, sblk],
        out_shape=[jax.ShapeDtypeStruct((N, D), jnp.float32),
                   jax.ShapeDtypeStruct((2, D), jnp.float32)],
    )(ah, parts)


def _norm_body(h_ref, pre_ref, st_ref, g_ref, b_ref, out_ref):
    mean = st_ref[0:1, :] * (1.0 / N)
    var = st_ref[1:2, :] * (1.0 / N) - mean * mean
    scale = lax.rsqrt(var + 1e-5) * g_ref[...]
    xn = (pre_ref[...] - mean) * scale + b_ref[...]
    out_ref[...] = h_ref[...] + jnp.maximum(xn, 0.0)


def _normalize(h, pre, st, gamma2, beta2):
    grid = (N // MM_BLK,)
    blk = pl.BlockSpec((MM_BLK, D), lambda i: (i, 0))
    sblk = pl.BlockSpec((2, D), lambda i: (0, 0))
    vec = pl.BlockSpec((1, D), lambda i: (0, 0))
    return pl.pallas_call(
        _norm_body,
        grid=grid,
        in_specs=[blk, blk, sblk, vec, vec],
        out_specs=blk,
        out_shape=jax.ShapeDtypeStruct((N, D), jnp.float32),
    )(h, pre, st, gamma2, beta2)


@functools.partial(jax.jit)
def kernel(h, edge_index, e, A_W, A_b, B_W, B_b, gamma, beta):
    ah, bh = _matmuls(h, A_W, A_b.reshape(1, D), B_W, B_b.reshape(1, D))

    npad = PAD_E - E
    src_p = jnp.concatenate([edge_index[0],
                             jnp.zeros((npad,), jnp.int32)])
    dst_p = jnp.concatenate([edge_index[1],
                             jnp.full((npad,), N, jnp.int32)])
    parts = _sc_aggregate(bh, src_p, dst_p)

    pre, st = _combine_stats(ah, parts)
    out = _normalize(h, pre, st, gamma.reshape(1, D), beta.reshape(1, D))
    return (out, e)


# speedup vs baseline: 3.7534x; 1.1928x over previous
import functools

import jax
import jax.numpy as jnp
from jax import lax
from jax.experimental import pallas as pl
from jax.experimental.pallas import tpu as pltpu
from jax.experimental.pallas import tpu_sc as plsc

N = 10000
E = 320000
D = 128

NC = 2
NS = 16
NW = NC * NS

CHUNK = 128
EPW = 10240
PAD_E = EPW * NW
ACC_ROWS = 10240
ROWS_PER_TILE = ACC_ROWS // NS
CP = 80

MM_BLK = 1000


def _mm_body(h_ref, aw_ref, ab_ref, bw_ref, bb_ref, ah_ref, bh_ref):
    hb = h_ref[...]
    ah_ref[...] = jnp.dot(hb, aw_ref[...],
                          preferred_element_type=jnp.float32) + ab_ref[...]
    bh_ref[...] = jnp.dot(hb, bw_ref[...],
                          preferred_element_type=jnp.float32) + bb_ref[...]


def _matmuls(h, A_W, A_b2, B_W, B_b2):
    grid = (N // MM_BLK,)
    full = pl.BlockSpec((D, D), lambda i: (0, 0))
    vec = pl.BlockSpec((1, D), lambda i: (0, 0))
    blk = pl.BlockSpec((MM_BLK, D), lambda i: (i, 0))
    return pl.pallas_call(
        _mm_body,
        grid=grid,
        in_specs=[blk, full, vec, full, vec],
        out_specs=[blk, blk],
        out_shape=[jax.ShapeDtypeStruct((N, D), jnp.float32)] * 2,
    )(h, A_W, A_b2, B_W, B_b2)


NB = 2
NCH = EPW // CHUNK
IB = 40


def _sc_body(bh, srcp, dstp, out, sidx, didx, rows, acc, gsem, ssem):
    c = lax.axis_index("c")
    s = lax.axis_index("s")
    wid = c * NS + s

    def _zb(i, _):
        rows[0, i // 8, pl.ds((i % 8) * 16, 16)] = jnp.zeros((16,),
                                                             jnp.float32)
        return _
    lax.fori_loop(0, CHUNK * 8, _zb, None)

    def _z(j, _):
        pltpu.sync_copy(rows.at[0],
                        acc.at[pl.ds(s * ROWS_PER_TILE + j * CHUNK, CHUNK)])
        return _
    lax.fori_loop(0, ROWS_PER_TILE // CHUNK, _z, None)

    plsc.subcore_barrier()

    for h in range(2):
        pltpu.sync_copy(srcp.at[wid, pl.ds(h * IB, IB)], sidx)
        pltpu.sync_copy(dstp.at[wid, pl.ds(h * IB, IB)], didx)

        for b in range(NB):
            pltpu.async_copy(bh.at[sidx.at[b]], rows.at[b], gsem.at[b])

        def _edge(k, _):
            for b in range(NB):
                kk = k * NB + b
                pltpu.make_async_copy(bh.at[sidx.at[kk]], rows.at[b],
                                      gsem.at[b]).wait()
                pltpu.async_copy(rows.at[b], acc.at[didx.at[kk]], ssem.at[b],
                                 add=True)
                bp = (b + NB - 1) % NB

                @pl.when((kk >= 1) & (kk + NB - 1 < IB))
                def _():
                    pltpu.make_async_copy(rows.at[bp], acc.at[didx.at[kk]],
                                          ssem.at[bp]).wait()
                    pltpu.async_copy(bh.at[sidx.at[kk + NB - 1]],
                                     rows.at[bp], gsem.at[bp])
            return _
        lax.fori_loop(0, IB // NB, _edge, None)

        for b in range(NB):
            pltpu.make_async_copy(rows.at[b],
                                  acc.at[didx.at[IB - NB + b]],
                                  ssem.at[b]).wait()

    plsc.subcore_barrier()

    def _cp(j, _):
        r0 = s * ROWS_PER_TILE + j * CHUNK
        pltpu.sync_copy(acc.at[pl.ds(r0, CHUNK)], rows.at[0])
        pltpu.sync_copy(rows.at[0], out.at[c, pl.ds(r0, CHUNK)])
        return _
    lax.fori_loop(0, ROWS_PER_TILE // CHUNK, _cp, None)


def _sc_aggregate(bh, src_p, dst_p):
    mesh = plsc.VectorSubcoreMesh(core_axis_name="c", subcore_axis_name="s")
    fn = pl.kernel(
        _sc_body,
        mesh=mesh,
        scratch_types=[
            pltpu.VMEM((IB, CHUNK), jnp.int32),
            pltpu.VMEM((IB, CHUNK), jnp.int32),
            pltpu.VMEM((NB, CHUNK, D), jnp.float32),
            pltpu.VMEM_SHARED((ACC_ROWS, D), jnp.float32),
            pltpu.SemaphoreType.DMA((NB,)),
            pltpu.SemaphoreType.DMA((NB,)),
        ],
        out_type=jax.ShapeDtypeStruct((NC, ACC_ROWS, D), jnp.float32),
    )
    return fn(bh, src_p, dst_p)


def _stats_body(ah_ref, p_ref, pre_ref, st_ref):
    i = pl.program_id(0)
    x = ah_ref[...] + p_ref[0] + p_ref[1]
    pre_ref[...] = x
    st = jnp.concatenate([jnp.sum(x, axis=0, keepdims=True),
                          jnp.sum(x * x, axis=0, keepdims=True)], axis=0)

    @pl.when(i == 0)
    def _():
        st_ref[...] = jnp.zeros_like(st_ref)

    st_ref[...] += st


def _combine_stats(ah, parts):
    grid = (N // MM_BLK,)
    blk = pl.BlockSpec((MM_BLK, D), lambda i: (i, 0))
    pblk = pl.BlockSpec((NC, MM_BLK, D), lambda i: (0, i, 0))
    sblk = pl.BlockSpec((2, D), lambda i: (0, 0))
    return pl.pallas_call(
        _stats_body,
        grid=grid,
        in_specs=[blk, pblk],
        out_specs=[blk, sblk],
        out_shape=[jax.ShapeDtypeStruct((N, D), jnp.float32),
                   jax.ShapeDtypeStruct((2, D), jnp.float32)],
    )(ah, parts)


def _norm_body(h_ref, pre_ref, st_ref, g_ref, b_ref, out_ref):
    mean = st_ref[0:1, :] * (1.0 / N)
    var = st_ref[1:2, :] * (1.0 / N) - mean * mean
    scale = lax.rsqrt(var + 1e-5) * g_ref[...]
    xn = (pre_ref[...] - mean) * scale + b_ref[...]
    out_ref[...] = h_ref[...] + jnp.maximum(xn, 0.0)


def _normalize(h, pre, st, gamma2, beta2):
    grid = (N // MM_BLK,)
    blk = pl.BlockSpec((MM_BLK, D), lambda i: (i, 0))
    sblk = pl.BlockSpec((2, D), lambda i: (0, 0))
    vec = pl.BlockSpec((1, D), lambda i: (0, 0))
    return pl.pallas_call(
        _norm_body,
        grid=grid,
        in_specs=[blk, blk, sblk, vec, vec],
        out_specs=blk,
        out_shape=jax.ShapeDtypeStruct((N, D), jnp.float32),
    )(h, pre, st, gamma2, beta2)


@functools.partial(jax.jit)
def kernel(h, edge_index, e, A_W, A_b, B_W, B_b, gamma, beta):
    ah, bh = _matmuls(h, A_W, A_b.reshape(1, D), B_W, B_b.reshape(1, D))

    npad = PAD_E - E
    src_p = jnp.concatenate([edge_index[0],
                             jnp.zeros((npad,), jnp.int32)])
    dst_p = jnp.concatenate([edge_index[1],
                             jnp.full((npad,), N, jnp.int32)])
    src_p = src_p.reshape(NW, NCH, CHUNK)
    dst_p = dst_p.reshape(NW, NCH, CHUNK)
    parts = _sc_aggregate(bh, src_p, dst_p)

    pre, st = _combine_stats(ah, parts)
    out = _normalize(h, pre, st, gamma.reshape(1, D), beta.reshape(1, D))
    return (out, e)


# speedup vs baseline: 9.2392x; 2.4615x over previous
import functools

import jax
import jax.numpy as jnp
from jax import lax
from jax.experimental import pallas as pl
from jax.experimental.pallas import tpu as pltpu
from jax.experimental.pallas import tpu_sc as plsc

N = 10000
E = 320000
D = 128

NC = 2
NS = 16
NW = NC * NS

CHUNK = 128
EPW = 10240
PAD_E = EPW * NW
ACC_ROWS = 10240
ROWS_PER_TILE = ACC_ROWS // NS
CP = 80

MM_BLK = 1000


def _mm_body(h_ref, aw_ref, ab_ref, bw_ref, bb_ref, ah_ref, bh_ref):
    hb = h_ref[...]
    ah_ref[...] = jnp.dot(hb, aw_ref[...],
                          preferred_element_type=jnp.float32) + ab_ref[...]
    bh_ref[...] = jnp.dot(hb, bw_ref[...],
                          preferred_element_type=jnp.float32) + bb_ref[...]


def _matmuls(h, A_W, A_b2, B_W, B_b2):
    grid = (N // MM_BLK,)
    full = pl.BlockSpec((D, D), lambda i: (0, 0))
    vec = pl.BlockSpec((1, D), lambda i: (0, 0))
    blk = pl.BlockSpec((MM_BLK, D), lambda i: (i, 0))
    return pl.pallas_call(
        _mm_body,
        grid=grid,
        in_specs=[blk, full, vec, full, vec],
        out_specs=[blk, blk],
        out_shape=[jax.ShapeDtypeStruct((N, D), jnp.float32)] * 2,
    )(h, A_W, A_b2, B_W, B_b2)


NB = 2
NCH = EPW // CHUNK
IB = 40


def _sc_body(bh, srcp, dstp, out, sidx, didx, rows, acc, gsem, ssem):
    c = lax.axis_index("c")
    s = lax.axis_index("s")
    wid = c * NS + s

    def _zb(i, _):
        rows[0, i // 8, pl.ds((i % 8) * 16, 16)] = jnp.zeros((16,),
                                                             jnp.float32)
        return _
    lax.fori_loop(0, CHUNK * 8, _zb, None)

    def _z(j, _):
        pltpu.sync_copy(rows.at[0],
                        acc.at[pl.ds(s * ROWS_PER_TILE + j * CHUNK, CHUNK)])
        return _
    lax.fori_loop(0, ROWS_PER_TILE // CHUNK, _z, None)

    plsc.subcore_barrier()

    for h in range(2):
        pltpu.sync_copy(srcp.at[wid, pl.ds(h * IB, IB)], sidx)
        pltpu.sync_copy(dstp.at[wid, pl.ds(h * IB, IB)], didx)

        for b in range(NB):
            pltpu.async_copy(bh.at[sidx.at[b]], rows.at[b], gsem.at[b])

        def _edge(k, _):
            for b in range(NB):
                kk = k * NB + b
                pltpu.make_async_copy(bh.at[sidx.at[kk]], rows.at[b],
                                      gsem.at[b]).wait()
                pltpu.async_copy(rows.at[b], acc.at[didx.at[kk]], ssem.at[b],
                                 add=True)
                bp = (b + NB - 1) % NB

                @pl.when((kk >= 1) & (kk + NB - 1 < IB))
                def _():
                    pltpu.make_async_copy(rows.at[bp], acc.at[didx.at[kk]],
                                          ssem.at[bp]).wait()
                    pltpu.async_copy(bh.at[sidx.at[kk + NB - 1]],
                                     rows.at[bp], gsem.at[bp])
            return _
        lax.fori_loop(0, IB // NB, _edge, None)

        for b in range(NB):
            pltpu.make_async_copy(rows.at[b],
                                  acc.at[didx.at[IB - NB + b]],
                                  ssem.at[b]).wait()

    plsc.subcore_barrier()

    def _cp(j, _):
        r0 = s * ROWS_PER_TILE + j * CHUNK
        pltpu.sync_copy(acc.at[pl.ds(r0, CHUNK)], rows.at[0])
        pltpu.sync_copy(rows.at[0], out.at[c, pl.ds(r0, CHUNK)])
        return _
    lax.fori_loop(0, ROWS_PER_TILE // CHUNK, _cp, None)


def _sc_aggregate(bh, src_p, dst_p):
    mesh = plsc.VectorSubcoreMesh(core_axis_name="c", subcore_axis_name="s")
    fn = pl.kernel(
        _sc_body,
        mesh=mesh,
        scratch_types=[
            pltpu.VMEM((IB, CHUNK), jnp.int32),
            pltpu.VMEM((IB, CHUNK), jnp.int32),
            pltpu.VMEM((NB, CHUNK, D), jnp.float32),
            pltpu.VMEM_SHARED((ACC_ROWS, D), jnp.float32),
            pltpu.SemaphoreType.DMA((NB,)),
            pltpu.SemaphoreType.DMA((NB,)),
        ],
        out_type=jax.ShapeDtypeStruct((NC, ACC_ROWS, D), jnp.float32),
    )
    return fn(bh, src_p, dst_p)


def _stats_body(ah_ref, p_ref, pre_ref, st_ref):
    i = pl.program_id(0)
    x = ah_ref[...] + p_ref[0] + p_ref[1]
    pre_ref[...] = x
    st = jnp.concatenate([jnp.sum(x, axis=0, keepdims=True),
                          jnp.sum(x * x, axis=0, keepdims=True)], axis=0)

    @pl.when(i == 0)
    def _():
        st_ref[...] = jnp.zeros_like(st_ref)

    st_ref[...] += st


def _combine_stats(ah, parts):
    grid = (N // MM_BLK,)
    blk = pl.BlockSpec((MM_BLK, D), lambda i: (i, 0))
    pblk = pl.BlockSpec((NC, MM_BLK, D), lambda i: (0, i, 0))
    sblk = pl.BlockSpec((2, D), lambda i: (0, 0))
    return pl.pallas_call(
        _stats_body,
        grid=grid,
        in_specs=[blk, pblk],
        out_specs=[blk, sblk],
        out_shape=[jax.ShapeDtypeStruct((N, D), jnp.float32),
                   jax.ShapeDtypeStruct((2, D), jnp.float32)],
    )(ah, parts)


def _norm_body(h_ref, pre_ref, st_ref, g_ref, b_ref, out_ref):
    mean = st_ref[0:1, :] * (1.0 / N)
    var = st_ref[1:2, :] * (1.0 / N) - mean * mean
    scale = lax.rsqrt(var + 1e-5) * g_ref[...]
    xn = (pre_ref[...] - mean) * scale + b_ref[...]
    out_ref[...] = h_ref[...] + jnp.maximum(xn, 0.0)


def _normalize(h, pre, st, gamma2, beta2):
    grid = (N // MM_BLK,)
    blk = pl.BlockSpec((MM_BLK, D), lambda i: (i, 0))
    sblk = pl.BlockSpec((2, D), lambda i: (0, 0))
    vec = pl.BlockSpec((1, D), lambda i: (0, 0))
    return pl.pallas_call(
        _norm_body,
        grid=grid,
        in_specs=[blk, blk, sblk, vec, vec],
        out_specs=blk,
        out_shape=jax.ShapeDtypeStruct((N, D), jnp.float32),
    )(h, pre, st, gamma2, beta2)


@functools.partial(jax.jit)
def kernel(h, edge_index, e, A_W, A_b, B_W, B_b, gamma, beta):
    ah, bh = _matmuls(h, A_W, A_b.reshape(1, D), B_W, B_b.reshape(1, D))

    npad = PAD_E - E
    pad_i = jnp.arange(npad, dtype=jnp.int32)
    src_p = jnp.concatenate([edge_index[0], pad_i % N])
    dst_p = jnp.concatenate([edge_index[1], N + pad_i % (ACC_ROWS - N)])
    src_p = src_p.reshape(NW, NCH, CHUNK)
    dst_p = dst_p.reshape(NW, NCH, CHUNK)
    parts = _sc_aggregate(bh, src_p, dst_p)

    pre, st = _combine_stats(ah, parts)
    out = _normalize(h, pre, st, gamma.reshape(1, D), beta.reshape(1, D))
    return (out, e)


# speedup vs baseline: 9.6325x; 1.0426x over previous
import functools

import jax
import jax.numpy as jnp
from jax import lax
from jax.experimental import pallas as pl
from jax.experimental.pallas import tpu as pltpu
from jax.experimental.pallas import tpu_sc as plsc

N = 10000
E = 320000
D = 128

NC = 2
NS = 16
NW = NC * NS

CHUNK = 128
ACC_ROWS = 10240
ROWS_PER_TILE = ACC_ROWS // NS

MM_BLK = 1000


def _mm_body(h_ref, aw_ref, ab_ref, bw_ref, bb_ref, ah_ref, bh_ref):
    hb = h_ref[...]
    ah_ref[...] = jnp.dot(hb, aw_ref[...],
                          preferred_element_type=jnp.float32) + ab_ref[...]
    bh_ref[...] = jnp.dot(hb, bw_ref[...],
                          preferred_element_type=jnp.float32) + bb_ref[...]


def _matmuls(h, A_W, A_b2, B_W, B_b2):
    grid = (N // MM_BLK,)
    full = pl.BlockSpec((D, D), lambda i: (0, 0))
    vec = pl.BlockSpec((1, D), lambda i: (0, 0))
    blk = pl.BlockSpec((MM_BLK, D), lambda i: (i, 0))
    return pl.pallas_call(
        _mm_body,
        grid=grid,
        in_specs=[blk, full, vec, full, vec],
        out_specs=[blk, blk],
        out_shape=[jax.ShapeDtypeStruct((N, D), jnp.float32)] * 2,
    )(h, A_W, A_b2, B_W, B_b2)


NB = 2
EPW = 10240
PAD_E = EPW * NW
NCH = EPW // CHUNK
IB = 40


def _sc_body(bh, srcp, dstp, out, sidx, didx, rows, acc, gsem, ssem):
    c = lax.axis_index("c")
    s = lax.axis_index("s")
    wid = c * NS + s

    def _zb(i, _):
        rows[0, i // 8, pl.ds((i % 8) * 16, 16)] = jnp.zeros((16,),
                                                             jnp.float32)
        return _
    lax.fori_loop(0, CHUNK * 8, _zb, None)

    def _z(j, _):
        pltpu.sync_copy(rows.at[0],
                        acc.at[pl.ds(s * ROWS_PER_TILE + j * CHUNK, CHUNK)])
        return _
    lax.fori_loop(0, ROWS_PER_TILE // CHUNK, _z, None)

    plsc.subcore_barrier()

    for h in range(2):
        pltpu.sync_copy(srcp.at[wid, pl.ds(h * IB, IB)], sidx)
        pltpu.sync_copy(dstp.at[wid, pl.ds(h * IB, IB)], didx)

        for b in range(NB):
            pltpu.async_copy(bh.at[sidx.at[b]], rows.at[b], gsem.at[b])

        def _edge(k, _):
            for b in range(NB):
                kk = k * NB + b
                pltpu.make_async_copy(bh.at[sidx.at[kk]], rows.at[b],
                                      gsem.at[b]).wait()
                pltpu.async_copy(rows.at[b], acc.at[didx.at[kk]], ssem.at[b],
                                 add=True)
                bp = (b + NB - 1) % NB

                @pl.when((kk >= 1) & (kk + NB - 1 < IB))
                def _():
                    pltpu.make_async_copy(rows.at[bp], acc.at[didx.at[kk]],
                                          ssem.at[bp]).wait()
                    pltpu.async_copy(bh.at[sidx.at[kk + NB - 1]],
                                     rows.at[bp], gsem.at[bp])
            return _
        lax.fori_loop(0, IB // NB, _edge, None)

        for b in range(NB):
            pltpu.make_async_copy(rows.at[b],
                                  acc.at[didx.at[IB - NB + b]],
                                  ssem.at[b]).wait()

    plsc.subcore_barrier()

    def _cp(j, _):
        r0 = s * ROWS_PER_TILE + j * CHUNK
        pltpu.sync_copy(acc.at[pl.ds(r0, CHUNK)], rows.at[0])
        pltpu.sync_copy(rows.at[0], out.at[c, pl.ds(r0, CHUNK)])
        return _
    lax.fori_loop(0, ROWS_PER_TILE // CHUNK, _cp, None)


def _sc_aggregate(bh, src_p, dst_p):
    mesh = plsc.VectorSubcoreMesh(core_axis_name="c", subcore_axis_name="s")
    fn = pl.kernel(
        _sc_body,
        mesh=mesh,
        scratch_types=[
            pltpu.VMEM((IB, CHUNK), jnp.int32),
            pltpu.VMEM((IB, CHUNK), jnp.int32),
            pltpu.VMEM((NB, CHUNK, D), jnp.float32),
            pltpu.VMEM_SHARED((ACC_ROWS, D), jnp.float32),
            pltpu.SemaphoreType.DMA((NB,)),
            pltpu.SemaphoreType.DMA((NB,)),
        ],
        out_type=jax.ShapeDtypeStruct((NC, ACC_ROWS, D), jnp.float32),
    )
    return fn(bh, src_p, dst_p)


def _final_body(h_ref, ah_ref, p_ref, g_ref, b_ref, out_ref):
    x = ah_ref[...] + p_ref[0, :N] + p_ref[1, :N]
    mean = jnp.mean(x, axis=0, keepdims=True)
    var = jnp.mean(x * x, axis=0, keepdims=True) - mean * mean
    xn = (x - mean) * (lax.rsqrt(var + 1e-5) * g_ref[...]) + b_ref[...]
    out_ref[...] = h_ref[...] + jnp.maximum(xn, 0.0)


def _finalize(h, ah, parts, gamma2, beta2):
    full = pl.BlockSpec((N, D), lambda: (0, 0))
    pblk = pl.BlockSpec((NC, ACC_ROWS, D), lambda: (0, 0, 0))
    vec = pl.BlockSpec((1, D), lambda: (0, 0))
    return pl.pallas_call(
        _final_body,
        in_specs=[full, full, pblk, vec, vec],
        out_specs=full,
        out_shape=jax.ShapeDtypeStruct((N, D), jnp.float32),
    )(h, ah, parts, gamma2, beta2)


@functools.partial(jax.jit)
def kernel(h, edge_index, e, A_W, A_b, B_W, B_b, gamma, beta):
    ah, bh = _matmuls(h, A_W, A_b.reshape(1, D), B_W, B_b.reshape(1, D))
    npad = PAD_E - E
    pad_i = jnp.arange(npad, dtype=jnp.int32)
    src_p = jnp.concatenate([edge_index[0], pad_i % N])
    dst_p = jnp.concatenate([edge_index[1], N + pad_i % (ACC_ROWS - N)])
    parts = _sc_aggregate(bh, src_p.reshape(NW, NCH, CHUNK),
                          dst_p.reshape(NW, NCH, CHUNK))
    out = _finalize(h, ah, parts, gamma.reshape(1, D), beta.reshape(1, D))
    return (out, e)


# speedup vs baseline: 10.6209x; 1.1026x over previous
import functools

import jax
import jax.numpy as jnp
from jax import lax
from jax.experimental import pallas as pl
from jax.experimental.pallas import tpu as pltpu
from jax.experimental.pallas import tpu_sc as plsc

N = 10000
E = 320000
D = 128

NC = 2
NS = 16
NW = NC * NS

CHUNK = 128
ACC_ROWS = 10240
ROWS_PER_TILE = ACC_ROWS // NS

MM_BLK = 1000


def _mm_body(h_ref, aw_ref, ab_ref, bw_ref, bb_ref, ah_ref, bh_ref):
    hb = h_ref[...]
    ah_ref[...] = jnp.dot(hb, aw_ref[...],
                          preferred_element_type=jnp.float32) + ab_ref[...]
    bh_ref[...] = jnp.dot(hb, bw_ref[...],
                          preferred_element_type=jnp.float32) + bb_ref[...]


def _matmuls(h, A_W, A_b2, B_W, B_b2):
    grid = (N // MM_BLK,)
    full = pl.BlockSpec((D, D), lambda i: (0, 0))
    vec = pl.BlockSpec((1, D), lambda i: (0, 0))
    blk = pl.BlockSpec((MM_BLK, D), lambda i: (i, 0))
    return pl.pallas_call(
        _mm_body,
        grid=grid,
        in_specs=[blk, full, vec, full, vec],
        out_specs=[blk, blk],
        out_shape=[jax.ShapeDtypeStruct((N, D), jnp.float32)] * 2,
    )(h, A_W, A_b2, B_W, B_b2)


NB = 2
NID = 4
NCHUNKS = E // CHUNK
WCH = 80
LASTW = NCHUNKS - (NW - 1) * WCH


def _sc_body(bh, ei, out, sidx, didx, rows, acc, gsem, ssem, isem):
    c = lax.axis_index("c")
    s = lax.axis_index("s")
    wid = c * NS + s

    def _zb(i, _):
        rows[0, i // 8, pl.ds((i % 8) * 16, 16)] = jnp.zeros((16,),
                                                             jnp.float32)
        return _
    lax.fori_loop(0, CHUNK * 8, _zb, None)

    def _z(j, _):
        pltpu.sync_copy(rows.at[0],
                        acc.at[pl.ds(s * ROWS_PER_TILE + j * CHUNK, CHUNK)])
        return _
    lax.fori_loop(0, ROWS_PER_TILE // CHUNK, _z, None)

    plsc.subcore_barrier()

    base = wid * WCH
    ncw = jnp.where(wid == NW - 1, LASTW, WCH)

    def _issue_idx(kk, sl):
        off = (base + kk) * CHUNK
        pltpu.async_copy(ei.at[0, pl.ds(off, CHUNK)], sidx.at[sl],
                         isem.at[sl])
        pltpu.async_copy(ei.at[1, pl.ds(off, CHUNK)], didx.at[sl],
                         isem.at[sl])

    def _wait_idx(kk, sl):
        off = (base + kk) * CHUNK
        pltpu.make_async_copy(ei.at[0, pl.ds(off, CHUNK)], sidx.at[sl],
                              isem.at[sl]).wait()
        pltpu.make_async_copy(ei.at[1, pl.ds(off, CHUNK)], didx.at[sl],
                              isem.at[sl]).wait()

    for t in range(3):
        _issue_idx(t, t)
    _wait_idx(0, 0)
    pltpu.async_copy(bh.at[sidx.at[0]], rows.at[0], gsem.at[0])

    def _edge(j, _):
        for b in range(NID):
            kk = j * NID + b
            r = b % NB
            rn = (b + 1) % NB
            sl_n = (b + 1) % NID
            sl_i = (b + 3) % NID
            pltpu.make_async_copy(bh.at[sidx.at[b]], rows.at[r],
                                  gsem.at[r]).wait()
            pltpu.async_copy(rows.at[r], acc.at[didx.at[b]], ssem.at[r],
                             add=True)

            @pl.when((kk >= 1) & (kk + 1 < ncw))
            def _():
                pltpu.make_async_copy(rows.at[rn], acc.at[didx.at[b]],
                                      ssem.at[rn]).wait()

            @pl.when(kk + 3 < ncw)
            def _():
                _issue_idx(kk + 3, sl_i)

            @pl.when(kk + 1 < ncw)
            def _():
                _wait_idx(kk + 1, sl_n)
                pltpu.async_copy(bh.at[sidx.at[sl_n]], rows.at[rn],
                                 gsem.at[rn])
        return _
    lax.fori_loop(0, ncw // NID, _edge, None)

    for t in range(NB):
        pltpu.make_async_copy(rows.at[t], acc.at[didx.at[t]],
                              ssem.at[t]).wait()

    plsc.subcore_barrier()

    def _cp(j, _):
        r0 = s * ROWS_PER_TILE + j * CHUNK
        pltpu.sync_copy(acc.at[pl.ds(r0, CHUNK)], rows.at[0])
        pltpu.sync_copy(rows.at[0], out.at[c, pl.ds(r0, CHUNK)])
        return _
    lax.fori_loop(0, ROWS_PER_TILE // CHUNK, _cp, None)


def _sc_aggregate(bh, edge_index):
    mesh = plsc.VectorSubcoreMesh(core_axis_name="c", subcore_axis_name="s")
    fn = pl.kernel(
        _sc_body,
        mesh=mesh,
        scratch_types=[
            pltpu.VMEM((NID, CHUNK), jnp.int32),
            pltpu.VMEM((NID, CHUNK), jnp.int32),
            pltpu.VMEM((NB, CHUNK, D), jnp.float32),
            pltpu.VMEM_SHARED((ACC_ROWS, D), jnp.float32),
            pltpu.SemaphoreType.DMA((NB,)),
            pltpu.SemaphoreType.DMA((NB,)),
            pltpu.SemaphoreType.DMA((NID,)),
        ],
        out_type=jax.ShapeDtypeStruct((NC, ACC_ROWS, D), jnp.float32),
    )
    return fn(bh, edge_index)


def _final_body(h_ref, ah_ref, p_ref, g_ref, b_ref, out_ref):
    x = ah_ref[...] + p_ref[0, :N] + p_ref[1, :N]
    mean = jnp.mean(x, axis=0, keepdims=True)
    var = jnp.mean(x * x, axis=0, keepdims=True) - mean * mean
    xn = (x - mean) * (lax.rsqrt(var + 1e-5) * g_ref[...]) + b_ref[...]
    out_ref[...] = h_ref[...] + jnp.maximum(xn, 0.0)


def _finalize(h, ah, parts, gamma2, beta2):
    full = pl.BlockSpec((N, D), lambda: (0, 0))
    pblk = pl.BlockSpec((NC, ACC_ROWS, D), lambda: (0, 0, 0))
    vec = pl.BlockSpec((1, D), lambda: (0, 0))
    return pl.pallas_call(
        _final_body,
        in_specs=[full, full, pblk, vec, vec],
        out_specs=full,
        out_shape=jax.ShapeDtypeStruct((N, D), jnp.float32),
    )(h, ah, parts, gamma2, beta2)


@functools.partial(jax.jit)
def kernel(h, edge_index, e, A_W, A_b, B_W, B_b, gamma, beta):
    ah, bh = _matmuls(h, A_W, A_b.reshape(1, D), B_W, B_b.reshape(1, D))
    parts = _sc_aggregate(bh, edge_index)
    out = _finalize(h, ah, parts, gamma.reshape(1, D), beta.reshape(1, D))
    return (out, e)
